# baseline (speedup 1.0000x reference)
import functools

import jax
import jax.numpy as jnp
from jax import lax
from jax.experimental import pallas as pl
from jax.experimental.pallas import tpu as pltpu
from jax.experimental.pallas import tpu_sc as plsc

K = 20
CDIM = 3
EPS = 1e-5
TQ = 512


def _prep_body(pts_ref, at_ref, bt_ref, g_ref, q_ref):
    p = pts_ref[...]
    g_ref[...] = jnp.dot(p, at_ref[...], preferred_element_type=jnp.float32)
    q_ref[...] = jnp.dot(p, bt_ref[...], preferred_element_type=jnp.float32)


def _topk_body(n_points, cq_ref, ck_ref, idx_ref):
    b = pl.program_id(0)
    q = cq_ref[0]
    ck = ck_ref[0]
    mm = lax.dot_general(q, ck, (((1,), (1,)), ((), ())),
                         preferred_element_type=jnp.float32)
    sqq = jnp.sum(q * q, axis=1, keepdims=True)
    sqk = jnp.sum(ck * ck, axis=1)[None, :]
    d2 = (sqq + sqk) - 2.0 * mm
    dist = jnp.sqrt(jnp.maximum(d2, 0.0))
    iota = lax.broadcasted_iota(jnp.int32, dist.shape, 1)
    big = jnp.int32(n_points)
    m = jnp.min(dist, axis=1, keepdims=True)
    c = jnp.min(jnp.where(dist == m, iota, big), axis=1, keepdims=True)
    cols = [c]
    for _ in range(K - 1):
        live = (dist > m) | ((dist == m) & (iota > c))
        mn = jnp.min(jnp.where(live, dist, jnp.inf), axis=1, keepdims=True)
        cn = jnp.min(jnp.where((dist == mn) & ((mn > m) | (iota > c)),
                               iota, big), axis=1, keepdims=True)
        m, c = mn, cn
        cols.append(c)
    idx_ref[0] = jnp.concatenate(cols, axis=1) + b * n_points


def _sc_gather_body(n_workers, q_per_worker, q_chunk,
                    idx_hbm, g_hbm,
                    gmax_hbm, gmin_hbm, gsum_hbm, gsq_hbm,
                    idx_v, rows_v, omax_v, omin_v, osum_v, osq_v, sem):
    wid = lax.axis_index("s") * 2 + lax.axis_index("c")
    n_chunks = q_per_worker // q_chunk
    e_chunk = q_chunk * K
    n_dma = e_chunk // 128

    def chunk_body(ch, carry):
        qbase = wid * q_per_worker + ch * q_chunk
        pltpu.sync_copy(idx_hbm.at[pl.ds(qbase * K, e_chunk)], idx_v)
        copies = []
        for g in range(n_dma):
            copies.append(pltpu.async_copy(
                g_hbm.at[idx_v.at[pl.ds(g * 128, 128)]],
                rows_v.at[pl.ds(g * 128, 128)], sem))
        for cp in copies:
            cp.wait()

        def q_body(qi, carry2):
            base = qi * K
            for c in range(4):
                sl = pl.ds(c * 16, 16)
                v = rows_v[base, sl]
                amax = v
                amin = v
                asum = v
                asq = v * v
                for j in range(1, K):
                    v = rows_v[base + j, sl]
                    amax = jnp.maximum(amax, v)
                    amin = jnp.minimum(amin, v)
                    asum = asum + v
                    asq = asq + v * v
                omax_v[qi, sl] = amax
                omin_v[qi, sl] = amin
                osum_v[qi, sl] = asum
                osq_v[qi, sl] = asq
            return carry2

        lax.fori_loop(0, q_chunk, q_body, 0)
        pltpu.sync_copy(omax_v, gmax_hbm.at[pl.ds(qbase, q_chunk)])
        pltpu.sync_copy(omin_v, gmin_hbm.at[pl.ds(qbase, q_chunk)])
        pltpu.sync_copy(osum_v, gsum_hbm.at[pl.ds(qbase, q_chunk)])
        pltpu.sync_copy(osq_v, gsq_hbm.at[pl.ds(qbase, q_chunk)])
        return carry

    lax.fori_loop(0, n_chunks, chunk_body, 0)


def _stats_body(n_edges, gsum_ref, gsq_ref, q_ref, gamma_ref, beta_ref,
                s_ref, t_ref):
    qv = q_ref[...]
    gs = gsum_ref[...]
    sum_h = (jnp.sum(gs, axis=0, keepdims=True)
             + K * jnp.sum(qv, axis=0, keepdims=True))
    sum_h2 = jnp.sum(gsq_ref[...] + 2.0 * qv * gs + K * qv * qv,
                     axis=0, keepdims=True)
    mean = sum_h / n_edges
    var = sum_h2 / n_edges - mean * mean
    s = gamma_ref[...] * lax.rsqrt(var + EPS)
    s_ref[...] = s
    t_ref[...] = beta_ref[...] - mean * s


def _final_body(gmax_ref, gmin_ref, q_ref, s_ref, t_ref, out_ref):
    s = s_ref[...]
    t = t_ref[...]
    hsel = jnp.where(s >= 0.0, gmax_ref[...], gmin_ref[...])
    y = jnp.maximum((hsel + q_ref[...]) * s + t, 0.0)
    out_ref[0] = y.T


def _prep_call(pts_flat, at, bt):
    BN = pts_flat.shape[0]
    f32 = jnp.float32
    return pl.pallas_call(
        _prep_body,
        out_shape=[jax.ShapeDtypeStruct((BN, 64), f32),
                   jax.ShapeDtypeStruct((BN, 64), f32)],
    )(pts_flat, at, bt)


def _topk_call(coords):
    B, N, _ = coords.shape
    return pl.pallas_call(
        functools.partial(_topk_body, N),
        grid=(B, N // TQ),
        in_specs=[
            pl.BlockSpec((1, TQ, CDIM), lambda b, i: (b, i, 0)),
            pl.BlockSpec((1, N, CDIM), lambda b, i: (b, 0, 0)),
        ],
        out_specs=pl.BlockSpec((1, TQ, K), lambda b, i: (b, i, 0)),
        out_shape=jax.ShapeDtypeStruct((B, N, K), jnp.int32),
    )(coords, coords)


def _sc_call(idx_flat, g_proj):
    BN = g_proj.shape[0]
    f32 = jnp.float32
    n_workers = 32
    q_per_worker = BN // n_workers
    q_chunk = 32
    e_chunk = q_chunk * K
    mesh = plsc.VectorSubcoreMesh(core_axis_name="c", subcore_axis_name="s")
    return pl.kernel(
        functools.partial(_sc_gather_body, n_workers, q_per_worker, q_chunk),
        out_type=[jax.ShapeDtypeStruct((BN, 64), f32)] * 4,
        mesh=mesh,
        scratch_types=[
            pltpu.VMEM((e_chunk,), jnp.int32),
            pltpu.VMEM((e_chunk, 64), f32),
            pltpu.VMEM((q_chunk, 64), f32),
            pltpu.VMEM((q_chunk, 64), f32),
            pltpu.VMEM((q_chunk, 64), f32),
            pltpu.VMEM((q_chunk, 64), f32),
            pltpu.SemaphoreType.DMA,
        ],
        compiler_params=pltpu.CompilerParams(use_tc_tiling_on_sc=False),
    )(idx_flat, g_proj)


def _stats_call(gsum, gsq, q_proj, gamma, beta):
    BN = q_proj.shape[0]
    f32 = jnp.float32
    return pl.pallas_call(
        functools.partial(_stats_body, float(BN * K)),
        out_shape=[jax.ShapeDtypeStruct((1, 64), f32),
                   jax.ShapeDtypeStruct((1, 64), f32)],
    )(gsum, gsq, q_proj, gamma.reshape(1, 64), beta.reshape(1, 64))


def _final_call(gmax, gmin, q_proj, s, t, B, N):
    f32 = jnp.float32
    return pl.pallas_call(
        _final_body,
        grid=(B,),
        in_specs=[
            pl.BlockSpec((N, 64), lambda b: (b, 0)),
            pl.BlockSpec((N, 64), lambda b: (b, 0)),
            pl.BlockSpec((N, 64), lambda b: (b, 0)),
            pl.BlockSpec((1, 64), lambda b: (0, 0)),
            pl.BlockSpec((1, 64), lambda b: (0, 0)),
        ],
        out_specs=pl.BlockSpec((1, 64, N), lambda b: (b, 0, 0)),
        out_shape=jax.ShapeDtypeStruct((B, 64, N), f32),
    )(gmax, gmin, q_proj, s, t)


def kernel(pts, W, gamma, beta):
    B, N, _ = pts.shape
    BN = B * N

    coords = pts[..., :CDIM]
    wc_c = W[:, 0:3]
    wc_i = W[:, 3:6]
    wa_c = W[:, 6:8]
    wa_i = W[:, 8:10]
    at = jnp.concatenate([wc_c, wa_c], axis=1).T
    bt = jnp.concatenate([wc_i - wc_c, wa_i - wa_c], axis=1).T
    pts_flat = pts.reshape(BN, 5)

    g_proj, q_proj = _prep_call(pts_flat, at, bt)
    idx = _topk_call(coords)
    idx_flat = idx.reshape(BN * K)
    gmax, gmin, gsum, gsq = _sc_call(idx_flat, g_proj)
    s, t = _stats_call(gsum, gsq, q_proj, gamma, beta)
    return _final_call(gmax, gmin, q_proj, s, t, B, N)


# speedup vs baseline: 2.0598x; 2.0598x over previous
import functools

import jax
import jax.numpy as jnp
from jax import lax
from jax.experimental import pallas as pl
from jax.experimental.pallas import tpu as pltpu
from jax.experimental.pallas import tpu_sc as plsc

K = 20
CDIM = 3
EPS = 1e-5
TQ = 256


def _prep_body(pts_ref, at_ref, bt_ref, g_ref, q_ref):
    p = pts_ref[...]
    g_ref[...] = jnp.dot(p, at_ref[...], preferred_element_type=jnp.float32)
    q_ref[...] = jnp.dot(p, bt_ref[...], preferred_element_type=jnp.float32)


def _topk_body(n_points, cq_ref, ck_ref, idx_ref):
    b = pl.program_id(0)
    q = cq_ref[0]
    ck = ck_ref[0]
    mm = lax.dot_general(q, ck, (((1,), (1,)), ((), ())),
                         preferred_element_type=jnp.float32)
    sqq = jnp.sum(q * q, axis=1, keepdims=True)
    sqk = jnp.sum(ck * ck, axis=1)[None, :]
    d2 = (sqq + sqk) - 2.0 * mm
    dist = jnp.sqrt(jnp.maximum(d2, 0.0))
    iota_f = lax.broadcasted_iota(jnp.int32, (n_points, 1), 0).astype(jnp.float32)
    bigf = jnp.float32(1e30)
    cols = []
    d = dist
    for _ in range(K):
        m = jnp.min(d, axis=1, keepdims=True)
        e = (d == m).astype(jnp.float32)
        c_f = jnp.dot(e, iota_f, preferred_element_type=jnp.float32)
        c = jnp.minimum(c_f, n_points - 1).astype(jnp.int32)
        cols.append(c)
        d = d + e * bigf
    idx_ref[0] = jnp.concatenate(cols, axis=1) + b * n_points


def _sc_gather_body(n_workers, q_per_worker, q_chunk,
                    idx_hbm, g_hbm,
                    gmax_hbm, gmin_hbm, gsum_hbm, gsq_hbm,
                    idx_v, rows_v, omax_v, omin_v, osum_v, osq_v, sem):
    wid = lax.axis_index("s") * 2 + lax.axis_index("c")
    n_chunks = q_per_worker // q_chunk
    e_chunk = q_chunk * K
    n_dma = e_chunk // 128

    def chunk_body(ch, carry):
        qbase = wid * q_per_worker + ch * q_chunk
        pltpu.sync_copy(idx_hbm.at[pl.ds(qbase * K, e_chunk)], idx_v)
        copies = []
        for g in range(n_dma):
            copies.append(pltpu.async_copy(
                g_hbm.at[idx_v.at[pl.ds(g * 128, 128)]],
                rows_v.at[pl.ds(g * 128, 128)], sem))
        for cp in copies:
            cp.wait()

        def q_body(qi, carry2):
            base = qi * K
            for c in range(4):
                sl = pl.ds(c * 16, 16)
                v = rows_v[base, sl]
                amax = v
                amin = v
                asum = v
                asq = v * v
                for j in range(1, K):
                    v = rows_v[base + j, sl]
                    amax = jnp.maximum(amax, v)
                    amin = jnp.minimum(amin, v)
                    asum = asum + v
                    asq = asq + v * v
                omax_v[qi, sl] = amax
                omin_v[qi, sl] = amin
                osum_v[qi, sl] = asum
                osq_v[qi, sl] = asq
            return carry2

        lax.fori_loop(0, q_chunk, q_body, 0)
        pltpu.sync_copy(omax_v, gmax_hbm.at[pl.ds(qbase, q_chunk)])
        pltpu.sync_copy(omin_v, gmin_hbm.at[pl.ds(qbase, q_chunk)])
        pltpu.sync_copy(osum_v, gsum_hbm.at[pl.ds(qbase, q_chunk)])
        pltpu.sync_copy(osq_v, gsq_hbm.at[pl.ds(qbase, q_chunk)])
        return carry

    lax.fori_loop(0, n_chunks, chunk_body, 0)


def _stats_body(n_edges, gsum_ref, gsq_ref, q_ref, gamma_ref, beta_ref,
                s_ref, t_ref):
    qv = q_ref[...]
    gs = gsum_ref[...]
    sum_h = (jnp.sum(gs, axis=0, keepdims=True)
             + K * jnp.sum(qv, axis=0, keepdims=True))
    sum_h2 = jnp.sum(gsq_ref[...] + 2.0 * qv * gs + K * qv * qv,
                     axis=0, keepdims=True)
    mean = sum_h / n_edges
    var = sum_h2 / n_edges - mean * mean
    s = gamma_ref[...] * lax.rsqrt(var + EPS)
    s_ref[...] = s
    t_ref[...] = beta_ref[...] - mean * s


def _final_body(gmax_ref, gmin_ref, q_ref, s_ref, t_ref, out_ref):
    s = s_ref[...]
    t = t_ref[...]
    hsel = jnp.where(s >= 0.0, gmax_ref[...], gmin_ref[...])
    y = jnp.maximum((hsel + q_ref[...]) * s + t, 0.0)
    out_ref[0] = y.T


def _prep_call(pts_flat, at, bt):
    BN = pts_flat.shape[0]
    f32 = jnp.float32
    return pl.pallas_call(
        _prep_body,
        out_shape=[jax.ShapeDtypeStruct((BN, 64), f32),
                   jax.ShapeDtypeStruct((BN, 64), f32)],
    )(pts_flat, at, bt)


def _topk_call(coords):
    B, N, _ = coords.shape
    return pl.pallas_call(
        functools.partial(_topk_body, N),
        grid=(B, N // TQ),
        in_specs=[
            pl.BlockSpec((1, TQ, CDIM), lambda b, i: (b, i, 0)),
            pl.BlockSpec((1, N, CDIM), lambda b, i: (b, 0, 0)),
        ],
        out_specs=pl.BlockSpec((1, TQ, K), lambda b, i: (b, i, 0)),
        out_shape=jax.ShapeDtypeStruct((B, N, K), jnp.int32),
    )(coords, coords)


def _sc_call(idx_flat, g_proj):
    BN = g_proj.shape[0]
    f32 = jnp.float32
    n_workers = 32
    q_per_worker = BN // n_workers
    q_chunk = 32
    e_chunk = q_chunk * K
    mesh = plsc.VectorSubcoreMesh(core_axis_name="c", subcore_axis_name="s")
    return pl.kernel(
        functools.partial(_sc_gather_body, n_workers, q_per_worker, q_chunk),
        out_type=[jax.ShapeDtypeStruct((BN, 64), f32)] * 4,
        mesh=mesh,
        scratch_types=[
            pltpu.VMEM((e_chunk,), jnp.int32),
            pltpu.VMEM((e_chunk, 64), f32),
            pltpu.VMEM((q_chunk, 64), f32),
            pltpu.VMEM((q_chunk, 64), f32),
            pltpu.VMEM((q_chunk, 64), f32),
            pltpu.VMEM((q_chunk, 64), f32),
            pltpu.SemaphoreType.DMA,
        ],
        compiler_params=pltpu.CompilerParams(use_tc_tiling_on_sc=False),
    )(idx_flat, g_proj)


def _stats_call(gsum, gsq, q_proj, gamma, beta):
    BN = q_proj.shape[0]
    f32 = jnp.float32
    return pl.pallas_call(
        functools.partial(_stats_body, float(BN * K)),
        out_shape=[jax.ShapeDtypeStruct((1, 64), f32),
                   jax.ShapeDtypeStruct((1, 64), f32)],
    )(gsum, gsq, q_proj, gamma.reshape(1, 64), beta.reshape(1, 64))


def _final_call(gmax, gmin, q_proj, s, t, B, N):
    f32 = jnp.float32
    return pl.pallas_call(
        _final_body,
        grid=(B,),
        in_specs=[
            pl.BlockSpec((N, 64), lambda b: (b, 0)),
            pl.BlockSpec((N, 64), lambda b: (b, 0)),
            pl.BlockSpec((N, 64), lambda b: (b, 0)),
            pl.BlockSpec((1, 64), lambda b: (0, 0)),
            pl.BlockSpec((1, 64), lambda b: (0, 0)),
        ],
        out_specs=pl.BlockSpec((1, 64, N), lambda b: (b, 0, 0)),
        out_shape=jax.ShapeDtypeStruct((B, 64, N), f32),
    )(gmax, gmin, q_proj, s, t)


def kernel(pts, W, gamma, beta):
    B, N, _ = pts.shape
    BN = B * N

    coords = pts[..., :CDIM]
    wc_c = W[:, 0:3]
    wc_i = W[:, 3:6]
    wa_c = W[:, 6:8]
    wa_i = W[:, 8:10]
    at = jnp.concatenate([wc_c, wa_c], axis=1).T
    bt = jnp.concatenate([wc_i - wc_c, wa_i - wa_c], axis=1).T
    pts_flat = pts.reshape(BN, 5)

    g_proj, q_proj = _prep_call(pts_flat, at, bt)
    idx = _topk_call(coords)
    idx_flat = idx.reshape(BN * K)
    gmax, gmin, gsum, gsq = _sc_call(idx_flat, g_proj)
    s, t = _stats_call(gsum, gsq, q_proj, gamma, beta)
    return _final_call(gmax, gmin, q_proj, s, t, B, N)
